# Initial kernel scaffold; baseline (speedup 1.0000x reference)
import jax
import jax.numpy as jnp
from jax.experimental import pallas as pl


def kernel(x, weight):
    raise NotImplementedError("write your pallas kernel here")


# speedup vs baseline: 1.4355x; 1.4355x over previous
import functools

import jax
import jax.numpy as jnp
from jax import lax
from jax.experimental import pallas as pl
from jax.experimental.pallas import tpu as pltpu
from jax.experimental.pallas import tpu_sc as plsc

DIM = 32
CHUNK = 128
NC = 2
NS = 16
NW = NC * NS


@functools.partial(jax.jit, static_argnums=(2,))
def _embed(idx2d, weight, n_chunks):
    c_per_w = n_chunks // NW
    B = n_chunks * CHUNK

    mesh = plsc.VectorSubcoreMesh(core_axis_name="c", subcore_axis_name="s")

    @functools.partial(
        pl.kernel,
        out_type=jax.ShapeDtypeStruct((B, DIM), jnp.float32),
        mesh=mesh,
        scratch_types=[
            pltpu.VMEM((c_per_w, CHUNK), jnp.int32),
            pltpu.VMEM((CHUNK, DIM), jnp.float32),
            pltpu.SemaphoreType.DMA,
        ],
        compiler_params=pltpu.CompilerParams(use_tc_tiling_on_sc=False),
    )
    def emb_kernel(idx_hbm, table_hbm, out_hbm, idx_v, rows_v, sem):
        wid = lax.axis_index("s") * NC + lax.axis_index("c")
        cbase = wid * c_per_w
        pltpu.sync_copy(idx_hbm.at[pl.ds(cbase, c_per_w)], idx_v)

        def body(j, carry):
            pltpu.async_copy(table_hbm.at[idx_v.at[j]], rows_v, sem).wait()
            pltpu.sync_copy(rows_v, out_hbm.at[pl.ds((cbase + j) * CHUNK, CHUNK)])
            return carry

        lax.fori_loop(0, c_per_w, body, 0)

    return emb_kernel(idx2d, weight)


def kernel(x, weight):
    bsz, fields = x.shape
    B = bsz * fields
    n_chunks = B // CHUNK
    idx2d = x.reshape(n_chunks, CHUNK).astype(jnp.int32)
    out = _embed(idx2d, weight, n_chunks)
    return out.reshape(bsz, fields, DIM)


# speedup vs baseline: 1.5740x; 1.0965x over previous
import functools

import jax
import jax.numpy as jnp
from jax import lax
from jax.experimental import pallas as pl
from jax.experimental.pallas import tpu as pltpu
from jax.experimental.pallas import tpu_sc as plsc

DIM = 32
CHUNK = 128
NC = 2
NS = 16
NW = NC * NS


NBUF = 8


@functools.partial(jax.jit, static_argnums=(2,))
def _embed(idx2d, weight, n_chunks):
    c_per_w = n_chunks // NW
    B = n_chunks * CHUNK
    n_groups = c_per_w // NBUF

    mesh = plsc.VectorSubcoreMesh(core_axis_name="c", subcore_axis_name="s")

    @functools.partial(
        pl.kernel,
        out_type=jax.ShapeDtypeStruct((B, DIM), jnp.float32),
        mesh=mesh,
        scratch_types=[
            pltpu.VMEM((c_per_w, CHUNK), jnp.int32),
            [pltpu.VMEM((CHUNK, DIM), jnp.float32) for _ in range(NBUF)],
            [pltpu.SemaphoreType.DMA for _ in range(NBUF)],
            [pltpu.SemaphoreType.DMA for _ in range(NBUF)],
        ],
        compiler_params=pltpu.CompilerParams(use_tc_tiling_on_sc=False),
    )
    def emb_kernel(idx_hbm, table_hbm, out_hbm, idx_v, bufs, gsems, ssems):
        wid = lax.axis_index("s") * NC + lax.axis_index("c")
        cbase = wid * c_per_w
        pltpu.sync_copy(idx_hbm.at[pl.ds(cbase, c_per_w)], idx_v)

        for b in range(NBUF):
            pltpu.async_copy(table_hbm.at[idx_v.at[b]], bufs[b], gsems[b])

        def group(g, carry):
            j0 = g * NBUF
            for b in range(NBUF):
                pltpu.make_async_copy(
                    table_hbm.at[idx_v.at[j0 + b]], bufs[b], gsems[b]
                ).wait()
                pltpu.async_copy(
                    bufs[b],
                    out_hbm.at[pl.ds((cbase + j0 + b) * CHUNK, CHUNK)],
                    ssems[b],
                )
            for b in range(NBUF):
                pltpu.make_async_copy(
                    bufs[b],
                    out_hbm.at[pl.ds((cbase + j0 + b) * CHUNK, CHUNK)],
                    ssems[b],
                ).wait()

                @pl.when(g + 1 < n_groups)
                def _():
                    pltpu.async_copy(
                        table_hbm.at[idx_v.at[j0 + NBUF + b]], bufs[b], gsems[b]
                    )

            return carry

        lax.fori_loop(0, n_groups, group, 0)

    return emb_kernel(idx2d, weight)


def kernel(x, weight):
    bsz, fields = x.shape
    B = bsz * fields
    n_chunks = B // CHUNK
    idx2d = x.reshape(n_chunks, CHUNK).astype(jnp.int32)
    out = _embed(idx2d, weight, n_chunks)
    return out.reshape(bsz, fields, DIM)
